# baseline (speedup 1.0000x reference)
import functools

import jax
import jax.numpy as jnp
from jax import lax
from jax.experimental import pallas as pl
from jax.experimental.pallas import tpu as pltpu
from jax.experimental.pallas import tpu_sc as plsc

_NC = 2
_NS = 16
_NW = _NC * _NS
_BB = 128
_NSLOT = 3


def _gather_call(S, B, D, units_per_w):
    mesh = plsc.VectorSubcoreMesh(core_axis_name="c", subcore_axis_name="s")
    DT = D // 8
    CB = B // _BB

    @functools.partial(
        pl.kernel,
        out_type=jax.ShapeDtypeStruct((S, DT, CB, 8, _BB), jnp.float32),
        mesh=mesh,
        compiler_params=pltpu.CompilerParams(
            use_tc_tiling_on_sc=False, needs_layout_passes=False
        ),
        scratch_types=[
            [pltpu.VMEM((_BB,), jnp.int32)] * _NSLOT,
            [pltpu.VMEM((_BB, D), jnp.float32)] * _NSLOT,
            [pltpu.VMEM((DT, 8, _BB + 1), jnp.float32)] * _NSLOT,
            [pltpu.SemaphoreType.DMA] * _NSLOT,
            [pltpu.SemaphoreType.DMA] * _NSLOT,
            [pltpu.SemaphoreType.DMA] * _NSLOT,
        ],
    )
    def k2(idx_hbm, table_hbm, out_hbm, idxs, gbufs, obufs, isems, gsems, wsems):
        wid = lax.axis_index("s") * _NC + lax.axis_index("c")
        u0 = wid * units_per_w

        def unit_sc(u):
            return u // CB, u % CB

        def idx_start(u, p):
            s, c = unit_sc(u)
            pltpu.async_copy(
                idx_hbm.at[s, pl.ds(c * _BB, _BB)], idxs[p], isems[p]
            )

        def gather_start(u, p):
            pltpu.make_async_copy(
                idx_hbm.at[0, pl.ds(0, _BB)], idxs[p], isems[p]
            ).wait()
            pltpu.async_copy(table_hbm.at[idxs[p]], gbufs[p], gsems[p])

        def write_start(u, p):
            s, c = unit_sc(u)
            pltpu.async_copy(
                obufs[p].at[:, :, pl.ds(0, _BB)],
                out_hbm.at[s, :, c, :, :],
                wsems[p],
            )

        def write_wait(p):
            pltpu.make_async_copy(
                obufs[p].at[:, :, pl.ds(0, _BB)],
                out_hbm.at[0, :, 0, :, :],
                wsems[p],
            ).wait()

        def transform(p):
            pltpu.make_async_copy(
                table_hbm.at[pl.ds(0, _BB), :], gbufs[p], gsems[p]
            ).wait()
            lanes = lax.iota(jnp.int32, 16)
            i0 = [(lanes + 16 * kk) >> 3 for kk in range(D // 16)]
            i1 = [(lanes + 16 * kk) & 7 for kk in range(D // 16)]
            pairs = [(kk, bb) for kk in range(D // 16) for bb in range(_BB)]
            for i in range(0, len(pairs), 16):
                chunk = pairs[i : i + 16]
                vals = [
                    gbufs[p][bb, pl.ds(16 * kk, 16)] for (kk, bb) in chunk
                ]
                for (kk, bb), v in zip(chunk, vals):
                    plsc.store_scatter(
                        obufs[p],
                        [i0[kk], i1[kk], jnp.full((16,), bb, jnp.int32)],
                        v,
                    )

        for p in range(_NSLOT):
            idx_start(u0 + p, p)
        gather_start(u0, 0)
        gather_start(u0 + 1, 1)

        n_iter = (units_per_w - 2) // _NSLOT

        def step(h, carry):
            g = h * _NSLOT
            for b in range(_NSLOT):
                u = u0 + g + b
                p = b
                gather_start(u + 2, (b + 2) % _NSLOT)

                @pl.when(g + b >= _NSLOT)
                def _():
                    write_wait(p)

                transform(p)
                write_start(u, p)

                @pl.when(g + b + _NSLOT < units_per_w)
                def _():
                    idx_start(u + _NSLOT, p)

            return carry

        lax.fori_loop(0, n_iter, step, 0)

        for r in range(_NSLOT * n_iter, units_per_w):
            u = u0 + r
            p = r % _NSLOT
            write_wait(p)
            transform(p)
            write_start(u, p)
        for p in range(_NSLOT):
            write_wait(p)

    return k2


def kernel(token_ids, weight):
    B, S = token_ids.shape
    V, D = weight.shape
    units = S * (B // _BB)
    units_per_w = units // _NW

    idx_t = token_ids.T.astype(jnp.int32)
    o5 = _gather_call(S, B, D, units_per_w)(idx_t, weight)
    return o5.transpose(2, 4, 0, 1, 3).reshape(B, S, D)
